# Initial kernel scaffold; baseline (speedup 1.0000x reference)
import jax
import jax.numpy as jnp
from jax.experimental import pallas as pl


def kernel(x, bert_x, edge_index, edge_type, pre_W, pre_b, ie_W1, ie_b1, ie_g1, ie_bb1, ie_W2, ie_b2, ln_g0, ln_b0, basis0, comp0, root0, bias0, ln_g1, ln_b1, basis1, comp1, root1, bias1, post_W, post_b):
    raise NotImplementedError("write your pallas kernel here")


# speedup vs baseline: 7.2402x; 7.2402x over previous
import jax
import jax.numpy as jnp
from jax import lax
from jax.experimental import pallas as pl
from jax.experimental.pallas import tpu as pltpu
from jax.experimental.pallas import tpu_sc as plsc

N = 10000
E = 320000
R = 8
NB = 4
D_IN = 128
H = 256
BERT = 768
IE = H + BERT

TS = 320
NW = 32
CAP = E + 80
NRPAD = 80128
DUMMYP = TS << 17
DUMMYR = N * R
SCAN = 16000
RB = 400
NRB = N // RB

_SC_MESH = plsc.VectorSubcoreMesh(core_axis_name="c", subcore_axis_name="s")
_SC_PARAMS = pltpu.CompilerParams(needs_layout_passes=False)


def _fuse_body(x_ref, bert_ref, preW_ref, preb_ref, W1_ref, b1_ref, g1_ref,
               bb1_ref, W2_ref, b2_ref, out_ref):
    h = jnp.dot(x_ref[...], preW_ref[...], preferred_element_type=jnp.float32)
    h = jnp.maximum(h + preb_ref[...], 0.0)
    z = jnp.concatenate([h, bert_ref[...]], axis=1)
    u = jnp.dot(z, W1_ref[...], preferred_element_type=jnp.float32) + b1_ref[...]
    m = jnp.mean(u, axis=1, keepdims=True)
    v = jnp.mean((u - m) ** 2, axis=1, keepdims=True)
    u = (u - m) * lax.rsqrt(v + 1e-5) * g1_ref[...] + bb1_ref[...]
    u = jnp.maximum(u, 0.0)
    w = jnp.tanh(jnp.dot(u, W2_ref[...], preferred_element_type=jnp.float32)
                 + b2_ref[...])
    out_ref[...] = w[:, :H]


def _fuse(x, bert_x, preW, preb, W1, b1, g1, bb1, W2, b2):
    return pl.pallas_call(
        _fuse_body,
        grid=(NRB,),
        in_specs=[
            pl.BlockSpec((RB, D_IN), lambda i: (i, 0)),
            pl.BlockSpec((RB, BERT), lambda i: (i, 0)),
            pl.BlockSpec((D_IN, H), lambda i: (0, 0)),
            pl.BlockSpec((1, H), lambda i: (0, 0)),
            pl.BlockSpec((IE, IE), lambda i: (0, 0)),
            pl.BlockSpec((1, IE), lambda i: (0, 0)),
            pl.BlockSpec((1, IE), lambda i: (0, 0)),
            pl.BlockSpec((1, IE), lambda i: (0, 0)),
            pl.BlockSpec((IE, IE), lambda i: (0, 0)),
            pl.BlockSpec((1, IE), lambda i: (0, 0)),
        ],
        out_specs=pl.BlockSpec((RB, H), lambda i: (i, 0)),
        out_shape=jax.ShapeDtypeStruct((N, H), jnp.float32),
    )(x, bert_x, preW, preb.reshape(1, H), W1, b1.reshape(1, IE),
      g1.reshape(1, IE), bb1.reshape(1, IE), W2, b2.reshape(1, IE))


def _prep_body(c0_ref, b0_ref, c1_ref, b1_ref, cnt_ref, W0_ref, W1_ref,
               inv_ref):
    W0_ref[...] = jnp.dot(c0_ref[...], b0_ref[...],
                          preferred_element_type=jnp.float32)
    W1_ref[...] = jnp.dot(c1_ref[...], b1_ref[...],
                          preferred_element_type=jnp.float32)
    c = jnp.sum(cnt_ref[...], axis=0, keepdims=True)
    col = lax.broadcasted_iota(jnp.int32, (1, NRPAD), 1)
    inv_ref[...] = jnp.where(col < N * R, 1.0 / jnp.maximum(c, 1.0), 0.0)


def _prep(comp0, basis0f, comp1, basis1f, cnt):
    return pl.pallas_call(
        _prep_body,
        grid=(1,),
        in_specs=[
            pl.BlockSpec((R, NB), lambda i: (0, 0)),
            pl.BlockSpec((NB, H * H), lambda i: (0, 0)),
            pl.BlockSpec((R, NB), lambda i: (0, 0)),
            pl.BlockSpec((NB, H * H), lambda i: (0, 0)),
            pl.BlockSpec((NW, NRPAD), lambda i: (0, 0)),
        ],
        out_specs=[
            pl.BlockSpec((R, H * H), lambda i: (0, 0)),
            pl.BlockSpec((R, H * H), lambda i: (0, 0)),
            pl.BlockSpec((1, NRPAD), lambda i: (0, 0)),
        ],
        out_shape=[
            jax.ShapeDtypeStruct((R, H * H), jnp.float32),
            jax.ShapeDtypeStruct((R, H * H), jnp.float32),
            jax.ShapeDtypeStruct((1, NRPAD), jnp.float32),
        ],
    )(comp0, basis0f, comp1, basis1f, cnt)


def _lnrelu_body(h_ref, g_ref, b_ref, o_ref):
    h = h_ref[...]
    m = jnp.mean(h, axis=1, keepdims=True)
    v = jnp.mean((h - m) ** 2, axis=1, keepdims=True)
    o_ref[...] = jnp.maximum((h - m) * lax.rsqrt(v + 1e-5) * g_ref[...]
                             + b_ref[...], 0.0)


def _lnrelu(h, g, b):
    return pl.pallas_call(
        _lnrelu_body,
        grid=(NRB,),
        in_specs=[
            pl.BlockSpec((RB, H), lambda i: (i, 0)),
            pl.BlockSpec((1, H), lambda i: (0, 0)),
            pl.BlockSpec((1, H), lambda i: (0, 0)),
        ],
        out_specs=pl.BlockSpec((RB, H), lambda i: (i, 0)),
        out_shape=jax.ShapeDtypeStruct((N, H), jnp.float32),
    )(h, g.reshape(1, H), b.reshape(1, H))


def _xw_body(t_ref, W_ref, o_ref):
    o_ref[...] = jnp.dot(t_ref[...], W_ref[0],
                         preferred_element_type=jnp.float32)[None]


def _xw(t, W):
    return pl.pallas_call(
        _xw_body,
        grid=(NRB, R),
        in_specs=[
            pl.BlockSpec((RB, H), lambda i, r: (i, 0)),
            pl.BlockSpec((1, H, H), lambda i, r: (r, 0, 0)),
        ],
        out_specs=pl.BlockSpec((1, RB, H), lambda i, r: (r, i, 0)),
        out_shape=jax.ShapeDtypeStruct((R, N, H), jnp.float32),
    )(t, W)


def _combine_body(h_ref, agg_ref, t_ref, root_ref, bias_ref, o_ref):
    o_ref[...] = (h_ref[...] + agg_ref[...] + bias_ref[...]
                  + jnp.dot(t_ref[...], root_ref[...],
                            preferred_element_type=jnp.float32))


def _combine(h, agg, t, root, bias):
    return pl.pallas_call(
        _combine_body,
        grid=(NRB,),
        in_specs=[
            pl.BlockSpec((RB, H), lambda i: (i, 0)),
            pl.BlockSpec((RB, H), lambda i: (i, 0)),
            pl.BlockSpec((RB, H), lambda i: (i, 0)),
            pl.BlockSpec((H, H), lambda i: (0, 0)),
            pl.BlockSpec((1, H), lambda i: (0, 0)),
        ],
        out_specs=pl.BlockSpec((RB, H), lambda i: (i, 0)),
        out_shape=jax.ShapeDtypeStruct((N, H), jnp.float32),
    )(h, agg, t, root, bias.reshape(1, H))


def _post_body(h_ref, W_ref, b_ref, o_ref):
    o_ref[...] = jnp.dot(h_ref[...], W_ref[...],
                         preferred_element_type=jnp.float32) + b_ref[...]


def _post(h, W, b):
    return pl.pallas_call(
        _post_body,
        grid=(NRB,),
        in_specs=[
            pl.BlockSpec((RB, H), lambda i: (i, 0)),
            pl.BlockSpec((H, H), lambda i: (0, 0)),
            pl.BlockSpec((1, H), lambda i: (0, 0)),
        ],
        out_specs=pl.BlockSpec((RB, H), lambda i: (i, 0)),
        out_shape=jax.ShapeDtypeStruct((N, H), jnp.float32),
    )(h, W, b.reshape(1, H))


def _wid():
    return lax.axis_index("c") * 16 + lax.axis_index("s")


def _counts_body(dst_hbm, typ_hbm, out_hbm, dbuf, tbuf, hist):
    w = _wid()
    zero = jnp.zeros((16,), jnp.float32)
    lane = lax.iota(jnp.int32, 16)

    def zf(i, _):
        hist[pl.ds(i * 16, 16)] = zero
        return 0

    lax.fori_loop(0, NRPAD // 16, zf, 0)
    ept = E // NW
    pltpu.sync_copy(dst_hbm.at[pl.ds(w * ept, ept)], dbuf)
    pltpu.sync_copy(typ_hbm.at[pl.ds(w * ept, ept)], tbuf)

    def step(k, _):
        d16 = dbuf[pl.ds(k * 16, 16)]
        t16 = tbuf[pl.ds(k * 16, 16)]
        ridx = t16 * N + d16
        for b in range(16):
            idx = ridx[b]
            base = (idx >> 4) << 4
            onehot = jnp.where(lane == (idx & 15), 1.0, 0.0)
            plsc.addupdate(hist.at[pl.ds(base, 16)], onehot)
        return 0

    lax.fori_loop(0, ept // 16, step, 0)
    pltpu.sync_copy(hist, out_hbm.at[pl.ds(w * NRPAD, NRPAD)])


def _counts(dst, typ):
    f = pl.kernel(
        _counts_body,
        out_type=jax.ShapeDtypeStruct((NW * NRPAD,), jnp.float32),
        mesh=_SC_MESH,
        compiler_params=_SC_PARAMS,
        scratch_types=[
            pltpu.VMEM((E // NW,), jnp.int32),
            pltpu.VMEM((E // NW,), jnp.int32),
            pltpu.VMEM((NRPAD,), jnp.float32),
        ],
    )
    return f(dst, typ)


def _compact_body(src_hbm, dst_hbm, typ_hbm, cpack_hbm, cridx_hbm, nb_hbm,
                  sbuf, dbuf, tbuf, cbuf, rbuf, vbuf):
    w = _wid()
    lane = lax.iota(jnp.int32, 16)
    lane16 = lane + 16
    dp16 = jnp.full((16,), DUMMYP, jnp.int32)
    dr16 = jnp.full((16,), DUMMYR, jnp.int32)
    for k in range(6):
        cbuf[pl.ds(k * 16, 16)] = dp16
        rbuf[pl.ds(k * 16, 16)] = dr16

    def blockstep(B, carry):
        cnt0, blk0 = carry
        pltpu.sync_copy(src_hbm.at[pl.ds(B * SCAN, SCAN)], sbuf)
        pltpu.sync_copy(dst_hbm.at[pl.ds(B * SCAN, SCAN)], dbuf)
        pltpu.sync_copy(typ_hbm.at[pl.ds(B * SCAN, SCAN)], tbuf)

        def chunk(k, carry2):
            cnt, blk = carry2
            s16 = sbuf[pl.ds(k * 16, 16)]
            d16 = dbuf[pl.ds(k * 16, 16)]
            t16 = tbuf[pl.ds(k * 16, 16)]
            own = (d16 * 13108) >> 22
            m = own == w
            dloc = d16 - own * TS
            vp = jnp.where(m, (t16 * N + s16) + (dloc << 17), DUMMYP)
            vr = jnp.where(m, t16 * N + d16, DUMMYR)
            key = jnp.where(m, lane, lane16)
            sp = plsc.sort_key_val(key, vp)[1]
            sr = plsc.sort_key_val(key, vr)[1]
            cbuf[pl.ds(cnt, 16)] = sp
            rbuf[pl.ds(cnt, 16)] = sr
            pc = plsc.all_reduce_population_count(m)
            cnt2 = cnt + pc[0]
            full = cnt2 >= 80

            @pl.when(full)
            def _():
                pltpu.sync_copy(
                    cbuf.at[pl.ds(0, 80)],
                    cpack_hbm.at[pl.ds(w * CAP + blk * 80, 80)])
                pltpu.sync_copy(
                    rbuf.at[pl.ds(0, 80)],
                    cridx_hbm.at[pl.ds(w * CAP + blk * 80, 80)])
                tv = cbuf[pl.ds(80, 16)]
                tr = rbuf[pl.ds(80, 16)]
                for k2 in range(6):
                    cbuf[pl.ds(k2 * 16, 16)] = dp16
                    rbuf[pl.ds(k2 * 16, 16)] = dr16
                cbuf[pl.ds(0, 16)] = tv
                rbuf[pl.ds(0, 16)] = tr

            cnt3 = jnp.where(full, cnt2 - 80, cnt2)
            blk2 = jnp.where(full, blk + 1, blk)
            return (cnt3, blk2)

        return lax.fori_loop(0, SCAN // 16, chunk, (cnt0, blk0))

    cnt, blk = lax.fori_loop(0, E // SCAN, blockstep,
                             (jnp.int32(0), jnp.int32(0)))

    @pl.when(cnt > 0)
    def _():
        cbuf[pl.ds(cnt, 16)] = dp16
        rbuf[pl.ds(cnt, 16)] = dr16
        pltpu.sync_copy(cbuf.at[pl.ds(0, 80)],
                        cpack_hbm.at[pl.ds(w * CAP + blk * 80, 80)])
        pltpu.sync_copy(rbuf.at[pl.ds(0, 80)],
                        cridx_hbm.at[pl.ds(w * CAP + blk * 80, 80)])

    nb = blk + jnp.where(cnt > 0, 1, 0).astype(jnp.int32)
    vbuf[pl.ds(0, 16)] = jnp.where(lane == 0, nb, 0)
    pltpu.sync_copy(vbuf, nb_hbm.at[pl.ds(w * 16, 16)])


def _compact(src, dst, typ):
    f = pl.kernel(
        _compact_body,
        out_type=[
            jax.ShapeDtypeStruct((NW * CAP,), jnp.int32),
            jax.ShapeDtypeStruct((NW * CAP,), jnp.int32),
            jax.ShapeDtypeStruct((NW * 16,), jnp.int32),
        ],
        mesh=_SC_MESH,
        compiler_params=_SC_PARAMS,
        scratch_types=[
            pltpu.VMEM((SCAN,), jnp.int32),
            pltpu.VMEM((SCAN,), jnp.int32),
            pltpu.VMEM((SCAN,), jnp.int32),
            pltpu.VMEM((96,), jnp.int32),
            pltpu.VMEM((96,), jnp.int32),
            pltpu.VMEM((16,), jnp.int32),
        ],
    )
    return f(src, dst, typ)


def _edge_body(cpack_hbm, cridx_hbm, nb_hbm, xw_hbm, inv_hbm, out_hbm,
               pbuf, ribuf, gbuf, ivals, rows, acc, nbb, sem):
    w = _wid()
    zero = jnp.zeros((16,), jnp.float32)

    def zf(i, _):
        for q in range(H // 16):
            acc[i, pl.ds(q * 16, 16)] = zero
        return 0

    lax.fori_loop(0, TS + 1, zf, 0)
    pltpu.sync_copy(nb_hbm.at[pl.ds(w * 16, 16)], nbb)
    nb = nbb[pl.ds(0, 16)][0]

    def batch(b, _):
        base = w * CAP + b * 80
        pltpu.sync_copy(cpack_hbm.at[pl.ds(base, 80)], pbuf)
        pltpu.sync_copy(cridx_hbm.at[pl.ds(base, 80)], ribuf)
        for a in range(5):
            sl = pl.ds(a * 16, 16)
            gbuf[sl] = pbuf[sl] & 131071
        pltpu.async_copy(xw_hbm.at[gbuf], rows, sem).wait()
        pltpu.async_copy(inv_hbm.at[ribuf], ivals, sem).wait()

        def acc5(a, _):
            vp16 = pbuf[pl.ds(a * 16, 16)]
            dl16 = vp16 >> 17
            iv16 = ivals[pl.ds(a * 16, 16)]
            for bb in range(16):
                j = dl16[bb]
                sp = jnp.full((16,), iv16[bb], jnp.float32)
                e = a * 16 + bb
                for q in range(H // 16):
                    sl = pl.ds(q * 16, 16)
                    plsc.addupdate(acc.at[j, sl], rows[e, sl] * sp)
            return 0

        lax.fori_loop(0, 5, acc5, 0)
        return 0

    lax.fori_loop(0, nb, batch, 0)

    @pl.when(w < NW - 1)
    def _():
        for q in range(4):
            pltpu.sync_copy(acc.at[pl.ds(q * 80, 80)],
                            out_hbm.at[pl.ds(w * TS + q * 80, 80)])

    @pl.when(w == NW - 1)
    def _():
        pltpu.sync_copy(acc.at[pl.ds(0, 80)],
                        out_hbm.at[pl.ds(w * TS, 80)])


def _edge(cpack, cridx, nbf, xwf, inv):
    f = pl.kernel(
        _edge_body,
        out_type=jax.ShapeDtypeStruct((N, H), jnp.float32),
        mesh=_SC_MESH,
        compiler_params=_SC_PARAMS,
        scratch_types=[
            pltpu.VMEM((80,), jnp.int32),
            pltpu.VMEM((80,), jnp.int32),
            pltpu.VMEM((80,), jnp.int32),
            pltpu.VMEM((80,), jnp.float32),
            pltpu.VMEM((80, H), jnp.float32),
            pltpu.VMEM((TS + 1, H), jnp.float32),
            pltpu.VMEM((16,), jnp.int32),
            pltpu.SemaphoreType.DMA,
        ],
    )
    return f(cpack, cridx, nbf, xwf, inv)


def kernel(x, bert_x, edge_index, edge_type, pre_W, pre_b, ie_W1, ie_b1,
           ie_g1, ie_bb1, ie_W2, ie_b2, ln_g0, ln_b0, basis0, comp0, root0,
           bias0, ln_g1, ln_b1, basis1, comp1, root1, bias1, post_W, post_b):
    src = edge_index[0]
    dst = edge_index[1]

    h = _fuse(x, bert_x, pre_W, pre_b, ie_W1, ie_b1, ie_g1, ie_bb1, ie_W2,
              ie_b2)
    cntp = _counts(dst, edge_type).reshape(NW, NRPAD)
    W0f, W1f, inv = _prep(comp0, basis0.reshape(NB, H * H), comp1,
                          basis1.reshape(NB, H * H), cntp)
    inv = inv.reshape(NRPAD)
    cpack, cridx, nbf = _compact(src, dst, edge_type)
    layers = [
        (ln_g0, ln_b0, W0f.reshape(R, H, H), root0, bias0),
        (ln_g1, ln_b1, W1f.reshape(R, H, H), root1, bias1),
    ]
    for (g, b, W, root, bias) in layers:
        t = _lnrelu(h, g, b)
        xwf = _xw(t, W).reshape(R * N, H)
        agg = _edge(cpack, cridx, nbf, xwf, inv)
        h = _combine(h, agg, t, root, bias)
    return _post(h, post_W, post_b)


# speedup vs baseline: 8.4274x; 1.1640x over previous
import jax
import jax.numpy as jnp
from jax import lax
from jax.experimental import pallas as pl
from jax.experimental.pallas import tpu as pltpu
from jax.experimental.pallas import tpu_sc as plsc

N = 10000
E = 320000
R = 8
NB = 4
D_IN = 128
H = 256
BERT = 768
IE = H + BERT

TS = 320
NW = 32
CAP = E + 80
NRPAD = 80128
DUMMYP = TS << 17
DUMMYR = N * R
SCAN = 16000
RB = 400
NRB = N // RB

_SC_MESH = plsc.VectorSubcoreMesh(core_axis_name="c", subcore_axis_name="s")
_SC_PARAMS = pltpu.CompilerParams(needs_layout_passes=False)


def _fuse_body(x_ref, bert_ref, preW_ref, preb_ref, W1_ref, b1_ref, g1_ref,
               bb1_ref, W2_ref, b2_ref, out_ref):
    h = jnp.dot(x_ref[...], preW_ref[...], preferred_element_type=jnp.float32)
    h = jnp.maximum(h + preb_ref[...], 0.0)
    z = jnp.concatenate([h, bert_ref[...]], axis=1)
    u = jnp.dot(z, W1_ref[...], preferred_element_type=jnp.float32) + b1_ref[...]
    m = jnp.mean(u, axis=1, keepdims=True)
    v = jnp.mean((u - m) ** 2, axis=1, keepdims=True)
    u = (u - m) * lax.rsqrt(v + 1e-5) * g1_ref[...] + bb1_ref[...]
    u = jnp.maximum(u, 0.0)
    w = jnp.tanh(jnp.dot(u, W2_ref[...], preferred_element_type=jnp.float32)
                 + b2_ref[...])
    out_ref[...] = w[:, :H]


def _fuse(x, bert_x, preW, preb, W1, b1, g1, bb1, W2, b2):
    return pl.pallas_call(
        _fuse_body,
        grid=(NRB,),
        in_specs=[
            pl.BlockSpec((RB, D_IN), lambda i: (i, 0)),
            pl.BlockSpec((RB, BERT), lambda i: (i, 0)),
            pl.BlockSpec((D_IN, H), lambda i: (0, 0)),
            pl.BlockSpec((1, H), lambda i: (0, 0)),
            pl.BlockSpec((IE, IE), lambda i: (0, 0)),
            pl.BlockSpec((1, IE), lambda i: (0, 0)),
            pl.BlockSpec((1, IE), lambda i: (0, 0)),
            pl.BlockSpec((1, IE), lambda i: (0, 0)),
            pl.BlockSpec((IE, IE), lambda i: (0, 0)),
            pl.BlockSpec((1, IE), lambda i: (0, 0)),
        ],
        out_specs=pl.BlockSpec((RB, H), lambda i: (i, 0)),
        out_shape=jax.ShapeDtypeStruct((N, H), jnp.float32),
    )(x, bert_x, preW, preb.reshape(1, H), W1, b1.reshape(1, IE),
      g1.reshape(1, IE), bb1.reshape(1, IE), W2, b2.reshape(1, IE))


def _prep_body(c0_ref, b0_ref, c1_ref, b1_ref, cnt_ref, W0_ref, W1_ref,
               inv_ref):
    W0_ref[...] = jnp.dot(c0_ref[...], b0_ref[...],
                          preferred_element_type=jnp.float32)
    W1_ref[...] = jnp.dot(c1_ref[...], b1_ref[...],
                          preferred_element_type=jnp.float32)
    c = jnp.sum(cnt_ref[...], axis=0, keepdims=True)
    col = lax.broadcasted_iota(jnp.int32, (1, NRPAD), 1)
    inv_ref[...] = jnp.where(col < N * R, 1.0 / jnp.maximum(c, 1.0), 0.0)


def _prep(comp0, basis0f, comp1, basis1f, cnt):
    return pl.pallas_call(
        _prep_body,
        grid=(1,),
        in_specs=[
            pl.BlockSpec((R, NB), lambda i: (0, 0)),
            pl.BlockSpec((NB, H * H), lambda i: (0, 0)),
            pl.BlockSpec((R, NB), lambda i: (0, 0)),
            pl.BlockSpec((NB, H * H), lambda i: (0, 0)),
            pl.BlockSpec((NW, NRPAD), lambda i: (0, 0)),
        ],
        out_specs=[
            pl.BlockSpec((R, H * H), lambda i: (0, 0)),
            pl.BlockSpec((R, H * H), lambda i: (0, 0)),
            pl.BlockSpec((1, NRPAD), lambda i: (0, 0)),
        ],
        out_shape=[
            jax.ShapeDtypeStruct((R, H * H), jnp.float32),
            jax.ShapeDtypeStruct((R, H * H), jnp.float32),
            jax.ShapeDtypeStruct((1, NRPAD), jnp.float32),
        ],
    )(comp0, basis0f, comp1, basis1f, cnt)


def _lnrelu_body(h_ref, g_ref, b_ref, o_ref):
    h = h_ref[...]
    m = jnp.mean(h, axis=1, keepdims=True)
    v = jnp.mean((h - m) ** 2, axis=1, keepdims=True)
    o_ref[...] = jnp.maximum((h - m) * lax.rsqrt(v + 1e-5) * g_ref[...]
                             + b_ref[...], 0.0)


def _lnrelu(h, g, b):
    return pl.pallas_call(
        _lnrelu_body,
        grid=(NRB,),
        in_specs=[
            pl.BlockSpec((RB, H), lambda i: (i, 0)),
            pl.BlockSpec((1, H), lambda i: (0, 0)),
            pl.BlockSpec((1, H), lambda i: (0, 0)),
        ],
        out_specs=pl.BlockSpec((RB, H), lambda i: (i, 0)),
        out_shape=jax.ShapeDtypeStruct((N, H), jnp.float32),
    )(h, g.reshape(1, H), b.reshape(1, H))


def _xw_body(t_ref, W_ref, o_ref):
    o_ref[...] = jnp.dot(t_ref[...], W_ref[0],
                         preferred_element_type=jnp.float32)[None]


def _xw(t, W):
    return pl.pallas_call(
        _xw_body,
        grid=(NRB, R),
        in_specs=[
            pl.BlockSpec((RB, H), lambda i, r: (i, 0)),
            pl.BlockSpec((1, H, H), lambda i, r: (r, 0, 0)),
        ],
        out_specs=pl.BlockSpec((1, RB, H), lambda i, r: (r, i, 0)),
        out_shape=jax.ShapeDtypeStruct((R, N, H), jnp.float32),
    )(t, W)


def _combine_body(h_ref, agg_ref, t_ref, root_ref, bias_ref, o_ref):
    o_ref[...] = (h_ref[...] + agg_ref[...] + bias_ref[...]
                  + jnp.dot(t_ref[...], root_ref[...],
                            preferred_element_type=jnp.float32))


def _combine(h, agg, t, root, bias):
    return pl.pallas_call(
        _combine_body,
        grid=(NRB,),
        in_specs=[
            pl.BlockSpec((RB, H), lambda i: (i, 0)),
            pl.BlockSpec((RB, H), lambda i: (i, 0)),
            pl.BlockSpec((RB, H), lambda i: (i, 0)),
            pl.BlockSpec((H, H), lambda i: (0, 0)),
            pl.BlockSpec((1, H), lambda i: (0, 0)),
        ],
        out_specs=pl.BlockSpec((RB, H), lambda i: (i, 0)),
        out_shape=jax.ShapeDtypeStruct((N, H), jnp.float32),
    )(h, agg, t, root, bias.reshape(1, H))


def _post_body(h_ref, W_ref, b_ref, o_ref):
    o_ref[...] = jnp.dot(h_ref[...], W_ref[...],
                         preferred_element_type=jnp.float32) + b_ref[...]


def _post(h, W, b):
    return pl.pallas_call(
        _post_body,
        grid=(NRB,),
        in_specs=[
            pl.BlockSpec((RB, H), lambda i: (i, 0)),
            pl.BlockSpec((H, H), lambda i: (0, 0)),
            pl.BlockSpec((1, H), lambda i: (0, 0)),
        ],
        out_specs=pl.BlockSpec((RB, H), lambda i: (i, 0)),
        out_shape=jax.ShapeDtypeStruct((N, H), jnp.float32),
    )(h, W, b.reshape(1, H))


def _wid():
    return lax.axis_index("c") * 16 + lax.axis_index("s")


def _counts_body(dst_hbm, typ_hbm, out_hbm, dbuf, tbuf, hist):
    w = _wid()
    zero = jnp.zeros((16,), jnp.float32)
    lane = lax.iota(jnp.int32, 16)

    def zf(i, _):
        hist[pl.ds(i * 16, 16)] = zero
        return 0

    lax.fori_loop(0, NRPAD // 16, zf, 0)
    ept = E // NW
    pltpu.sync_copy(dst_hbm.at[pl.ds(w * ept, ept)], dbuf)
    pltpu.sync_copy(typ_hbm.at[pl.ds(w * ept, ept)], tbuf)

    def step(k, _):
        d16 = dbuf[pl.ds(k * 16, 16)]
        t16 = tbuf[pl.ds(k * 16, 16)]
        ridx = t16 * N + d16
        for b in range(16):
            idx = ridx[b]
            base = (idx >> 4) << 4
            onehot = jnp.where(lane == (idx & 15), 1.0, 0.0)
            plsc.addupdate(hist.at[pl.ds(base, 16)], onehot)
        return 0

    lax.fori_loop(0, ept // 16, step, 0)
    pltpu.sync_copy(hist, out_hbm.at[pl.ds(w * NRPAD, NRPAD)])


def _counts(dst, typ):
    f = pl.kernel(
        _counts_body,
        out_type=jax.ShapeDtypeStruct((NW * NRPAD,), jnp.float32),
        mesh=_SC_MESH,
        compiler_params=_SC_PARAMS,
        scratch_types=[
            pltpu.VMEM((E // NW,), jnp.int32),
            pltpu.VMEM((E // NW,), jnp.int32),
            pltpu.VMEM((NRPAD,), jnp.float32),
        ],
    )
    return f(dst, typ)


def _compact_body(src_hbm, dst_hbm, typ_hbm, cpack_hbm, cridx_hbm, nb_hbm,
                  sbuf, dbuf, tbuf, cbuf, rbuf, vbuf):
    w = _wid()
    lane = lax.iota(jnp.int32, 16)
    lane16 = lane + 16
    dp16 = jnp.full((16,), DUMMYP, jnp.int32)
    dr16 = jnp.full((16,), DUMMYR, jnp.int32)
    for k in range(6):
        cbuf[pl.ds(k * 16, 16)] = dp16
        rbuf[pl.ds(k * 16, 16)] = dr16

    def blockstep(B, carry):
        cnt0, blk0 = carry
        pltpu.sync_copy(src_hbm.at[pl.ds(B * SCAN, SCAN)], sbuf)
        pltpu.sync_copy(dst_hbm.at[pl.ds(B * SCAN, SCAN)], dbuf)
        pltpu.sync_copy(typ_hbm.at[pl.ds(B * SCAN, SCAN)], tbuf)

        def chunk(k, carry2):
            cnt, blk = carry2
            s16 = sbuf[pl.ds(k * 16, 16)]
            d16 = dbuf[pl.ds(k * 16, 16)]
            t16 = tbuf[pl.ds(k * 16, 16)]
            own = (d16 * 13108) >> 22
            m = own == w
            dloc = d16 - own * TS
            vp = jnp.where(m, (t16 * N + s16) + (dloc << 17), DUMMYP)
            vr = jnp.where(m, t16 * N + d16, DUMMYR)
            key = jnp.where(m, lane, lane16)
            sp = plsc.sort_key_val(key, vp)[1]
            sr = plsc.sort_key_val(key, vr)[1]
            cbuf[pl.ds(cnt, 16)] = sp
            rbuf[pl.ds(cnt, 16)] = sr
            pc = plsc.all_reduce_population_count(m)
            cnt2 = cnt + pc[0]
            full = cnt2 >= 80

            @pl.when(full)
            def _():
                pltpu.sync_copy(
                    cbuf.at[pl.ds(0, 80)],
                    cpack_hbm.at[pl.ds(w * CAP + blk * 80, 80)])
                pltpu.sync_copy(
                    rbuf.at[pl.ds(0, 80)],
                    cridx_hbm.at[pl.ds(w * CAP + blk * 80, 80)])
                tv = cbuf[pl.ds(80, 16)]
                tr = rbuf[pl.ds(80, 16)]
                for k2 in range(6):
                    cbuf[pl.ds(k2 * 16, 16)] = dp16
                    rbuf[pl.ds(k2 * 16, 16)] = dr16
                cbuf[pl.ds(0, 16)] = tv
                rbuf[pl.ds(0, 16)] = tr

            cnt3 = jnp.where(full, cnt2 - 80, cnt2)
            blk2 = jnp.where(full, blk + 1, blk)
            return (cnt3, blk2)

        return lax.fori_loop(0, SCAN // 16, chunk, (cnt0, blk0))

    cnt, blk = lax.fori_loop(0, E // SCAN, blockstep,
                             (jnp.int32(0), jnp.int32(0)))

    @pl.when(cnt > 0)
    def _():
        cbuf[pl.ds(cnt, 16)] = dp16
        rbuf[pl.ds(cnt, 16)] = dr16
        pltpu.sync_copy(cbuf.at[pl.ds(0, 80)],
                        cpack_hbm.at[pl.ds(w * CAP + blk * 80, 80)])
        pltpu.sync_copy(rbuf.at[pl.ds(0, 80)],
                        cridx_hbm.at[pl.ds(w * CAP + blk * 80, 80)])

    nb = blk + jnp.where(cnt > 0, 1, 0).astype(jnp.int32)
    vbuf[pl.ds(0, 16)] = jnp.where(lane == 0, nb, 0)
    pltpu.sync_copy(vbuf, nb_hbm.at[pl.ds(w * 16, 16)])


def _compact(src, dst, typ):
    f = pl.kernel(
        _compact_body,
        out_type=[
            jax.ShapeDtypeStruct((NW * CAP,), jnp.int32),
            jax.ShapeDtypeStruct((NW * CAP,), jnp.int32),
            jax.ShapeDtypeStruct((NW * 16,), jnp.int32),
        ],
        mesh=_SC_MESH,
        compiler_params=_SC_PARAMS,
        scratch_types=[
            pltpu.VMEM((SCAN,), jnp.int32),
            pltpu.VMEM((SCAN,), jnp.int32),
            pltpu.VMEM((SCAN,), jnp.int32),
            pltpu.VMEM((96,), jnp.int32),
            pltpu.VMEM((96,), jnp.int32),
            pltpu.VMEM((16,), jnp.int32),
        ],
    )
    return f(src, dst, typ)


def _edge_body(cpack_hbm, cridx_hbm, nb_hbm, xw_hbm, inv_hbm, out_hbm,
               pbufA, ribufA, gbufA, ivalsA, rowsA,
               pbufB, ribufB, gbufB, ivalsB, rowsB,
               acc, nbb, semA, isemA, semB, isemB):
    w = _wid()
    zero = jnp.zeros((16,), jnp.float32)

    def zf(i, _):
        for q in range(H // 16):
            acc[i, pl.ds(q * 16, 16)] = zero
        return 0

    lax.fori_loop(0, TS + 1, zf, 0)
    pltpu.sync_copy(nb_hbm.at[pl.ds(w * 16, 16)], nbb)
    nb = nbb[pl.ds(0, 16)][0]

    bufsA = (pbufA, ribufA, gbufA, ivalsA, rowsA, semA, isemA)
    bufsB = (pbufB, ribufB, gbufB, ivalsB, rowsB, semB, isemB)

    def issue(b, bufs):
        pbuf, ribuf, gbuf, ivals, rows, sem, isem = bufs
        base = w * CAP + b * 80
        pltpu.sync_copy(cpack_hbm.at[pl.ds(base, 80)], pbuf)
        pltpu.sync_copy(cridx_hbm.at[pl.ds(base, 80)], ribuf)
        for a in range(5):
            sl = pl.ds(a * 16, 16)
            gbuf[sl] = pbuf[sl] & 131071
        pltpu.async_copy(xw_hbm.at[gbuf], rows, sem)
        pltpu.async_copy(inv_hbm.at[ribuf], ivals, isem)

    def consume(bufs):
        pbuf, ribuf, gbuf, ivals, rows, sem, isem = bufs
        pltpu.make_async_copy(xw_hbm.at[gbuf], rows, sem).wait()
        pltpu.make_async_copy(inv_hbm.at[ribuf], ivals, isem).wait()

        def acc5(a, _):
            vp16 = pbuf[pl.ds(a * 16, 16)]
            dl16 = vp16 >> 17
            iv16 = ivals[pl.ds(a * 16, 16)]
            for bb in range(16):
                j = dl16[bb]
                sp = jnp.full((16,), iv16[bb], jnp.float32)
                e = a * 16 + bb
                for q in range(H // 16):
                    sl = pl.ds(q * 16, 16)
                    plsc.addupdate(acc.at[j, sl], rows[e, sl] * sp)
            return 0

        lax.fori_loop(0, 5, acc5, 0)

    @pl.when(nb > 0)
    def _():
        issue(jnp.int32(0), bufsA)

    @pl.when(nb > 1)
    def _():
        issue(jnp.int32(1), bufsB)

    def batch(b, _):
        @pl.when((b & 1) == 0)
        def _():
            consume(bufsA)

            @pl.when(b + 2 < nb)
            def _():
                issue(b + 2, bufsA)

        @pl.when((b & 1) == 1)
        def _():
            consume(bufsB)

            @pl.when(b + 2 < nb)
            def _():
                issue(b + 2, bufsB)

        return 0

    lax.fori_loop(0, nb, batch, 0)

    @pl.when(w < NW - 1)
    def _():
        for q in range(4):
            pltpu.sync_copy(acc.at[pl.ds(q * 80, 80)],
                            out_hbm.at[pl.ds(w * TS + q * 80, 80)])

    @pl.when(w == NW - 1)
    def _():
        pltpu.sync_copy(acc.at[pl.ds(0, 80)],
                        out_hbm.at[pl.ds(w * TS, 80)])


def _edge(cpack, cridx, nbf, xwf, inv):
    f = pl.kernel(
        _edge_body,
        out_type=jax.ShapeDtypeStruct((N, H), jnp.float32),
        mesh=_SC_MESH,
        compiler_params=_SC_PARAMS,
        scratch_types=[
            pltpu.VMEM((80,), jnp.int32),
            pltpu.VMEM((80,), jnp.int32),
            pltpu.VMEM((80,), jnp.int32),
            pltpu.VMEM((80,), jnp.float32),
            pltpu.VMEM((80, H), jnp.float32),
            pltpu.VMEM((80,), jnp.int32),
            pltpu.VMEM((80,), jnp.int32),
            pltpu.VMEM((80,), jnp.int32),
            pltpu.VMEM((80,), jnp.float32),
            pltpu.VMEM((80, H), jnp.float32),
            pltpu.VMEM((TS + 1, H), jnp.float32),
            pltpu.VMEM((16,), jnp.int32),
            pltpu.SemaphoreType.DMA,
            pltpu.SemaphoreType.DMA,
            pltpu.SemaphoreType.DMA,
            pltpu.SemaphoreType.DMA,
        ],
    )
    return f(cpack, cridx, nbf, xwf, inv)


def kernel(x, bert_x, edge_index, edge_type, pre_W, pre_b, ie_W1, ie_b1,
           ie_g1, ie_bb1, ie_W2, ie_b2, ln_g0, ln_b0, basis0, comp0, root0,
           bias0, ln_g1, ln_b1, basis1, comp1, root1, bias1, post_W, post_b):
    src = edge_index[0]
    dst = edge_index[1]

    h = _fuse(x, bert_x, pre_W, pre_b, ie_W1, ie_b1, ie_g1, ie_bb1, ie_W2,
              ie_b2)
    cntp = _counts(dst, edge_type).reshape(NW, NRPAD)
    W0f, W1f, inv = _prep(comp0, basis0.reshape(NB, H * H), comp1,
                          basis1.reshape(NB, H * H), cntp)
    inv = inv.reshape(NRPAD)
    cpack, cridx, nbf = _compact(src, dst, edge_type)
    layers = [
        (ln_g0, ln_b0, W0f.reshape(R, H, H), root0, bias0),
        (ln_g1, ln_b1, W1f.reshape(R, H, H), root1, bias1),
    ]
    for (g, b, W, root, bias) in layers:
        t = _lnrelu(h, g, b)
        xwf = _xw(t, W).reshape(R * N, H)
        agg = _edge(cpack, cridx, nbf, xwf, inv)
        h = _combine(h, agg, t, root, bias)
    return _post(h, post_W, post_b)


# speedup vs baseline: 9.4085x; 1.1164x over previous
import jax
import jax.numpy as jnp
from jax import lax
from jax.experimental import pallas as pl
from jax.experimental.pallas import tpu as pltpu
from jax.experimental.pallas import tpu_sc as plsc

N = 10000
E = 320000
R = 8
NB = 4
D_IN = 128
H = 256
BERT = 768
IE = H + BERT

TS = 320
NW = 32
CAP = E + 80
NRPAD = 80128
DUMMYP = TS << 14
SCAN = 16000
RB = 400
NRB = N // RB

_SC_MESH = plsc.VectorSubcoreMesh(core_axis_name="c", subcore_axis_name="s")
_SC_PARAMS = pltpu.CompilerParams(needs_layout_passes=False)


def _fuse_body(x_ref, bert_ref, preW_ref, preb_ref, W1_ref, b1_ref, g1_ref,
               bb1_ref, W2_ref, b2_ref, out_ref):
    h = jnp.dot(x_ref[...], preW_ref[...], preferred_element_type=jnp.float32)
    h = jnp.maximum(h + preb_ref[...], 0.0)
    z = jnp.concatenate([h, bert_ref[...]], axis=1)
    u = jnp.dot(z, W1_ref[...], preferred_element_type=jnp.float32) + b1_ref[...]
    m = jnp.mean(u, axis=1, keepdims=True)
    v = jnp.mean((u - m) ** 2, axis=1, keepdims=True)
    u = (u - m) * lax.rsqrt(v + 1e-5) * g1_ref[...] + bb1_ref[...]
    u = jnp.maximum(u, 0.0)
    w = jnp.tanh(jnp.dot(u, W2_ref[...], preferred_element_type=jnp.float32)
                 + b2_ref[...])
    out_ref[...] = w[:, :H]


def _fuse(x, bert_x, preW, preb, W1, b1, g1, bb1, W2, b2):
    return pl.pallas_call(
        _fuse_body,
        grid=(NRB,),
        in_specs=[
            pl.BlockSpec((RB, D_IN), lambda i: (i, 0)),
            pl.BlockSpec((RB, BERT), lambda i: (i, 0)),
            pl.BlockSpec((D_IN, H), lambda i: (0, 0)),
            pl.BlockSpec((1, H), lambda i: (0, 0)),
            pl.BlockSpec((IE, IE), lambda i: (0, 0)),
            pl.BlockSpec((1, IE), lambda i: (0, 0)),
            pl.BlockSpec((1, IE), lambda i: (0, 0)),
            pl.BlockSpec((1, IE), lambda i: (0, 0)),
            pl.BlockSpec((IE, IE), lambda i: (0, 0)),
            pl.BlockSpec((1, IE), lambda i: (0, 0)),
        ],
        out_specs=pl.BlockSpec((RB, H), lambda i: (i, 0)),
        out_shape=jax.ShapeDtypeStruct((N, H), jnp.float32),
    )(x, bert_x, preW, preb.reshape(1, H), W1, b1.reshape(1, IE),
      g1.reshape(1, IE), bb1.reshape(1, IE), W2, b2.reshape(1, IE))


def _prep_body(c0_ref, b0_ref, c1_ref, b1_ref, cnt_ref, W0_ref, W1_ref,
               inv_ref):
    W0_ref[...] = jnp.dot(c0_ref[...], b0_ref[...],
                          preferred_element_type=jnp.float32)
    W1_ref[...] = jnp.dot(c1_ref[...], b1_ref[...],
                          preferred_element_type=jnp.float32)
    c = jnp.sum(cnt_ref[...], axis=0, keepdims=True)
    col = lax.broadcasted_iota(jnp.int32, (1, NRPAD), 1)
    inv_ref[...] = jnp.where(col < N * R, 1.0 / jnp.maximum(c, 1.0), 0.0)


def _prep(comp0, basis0f, comp1, basis1f, cnt):
    return pl.pallas_call(
        _prep_body,
        grid=(1,),
        in_specs=[
            pl.BlockSpec((R, NB), lambda i: (0, 0)),
            pl.BlockSpec((NB, H * H), lambda i: (0, 0)),
            pl.BlockSpec((R, NB), lambda i: (0, 0)),
            pl.BlockSpec((NB, H * H), lambda i: (0, 0)),
            pl.BlockSpec((NW, NRPAD), lambda i: (0, 0)),
        ],
        out_specs=[
            pl.BlockSpec((R, H * H), lambda i: (0, 0)),
            pl.BlockSpec((R, H * H), lambda i: (0, 0)),
            pl.BlockSpec((1, NRPAD), lambda i: (0, 0)),
        ],
        out_shape=[
            jax.ShapeDtypeStruct((R, H * H), jnp.float32),
            jax.ShapeDtypeStruct((R, H * H), jnp.float32),
            jax.ShapeDtypeStruct((1, NRPAD), jnp.float32),
        ],
    )(comp0, basis0f, comp1, basis1f, cnt)


def _lnrelu_body(h_ref, g_ref, b_ref, o_ref):
    h = h_ref[...]
    m = jnp.mean(h, axis=1, keepdims=True)
    v = jnp.mean((h - m) ** 2, axis=1, keepdims=True)
    o_ref[...] = jnp.maximum((h - m) * lax.rsqrt(v + 1e-5) * g_ref[...]
                             + b_ref[...], 0.0)


def _lnrelu(h, g, b):
    return pl.pallas_call(
        _lnrelu_body,
        grid=(NRB,),
        in_specs=[
            pl.BlockSpec((RB, H), lambda i: (i, 0)),
            pl.BlockSpec((1, H), lambda i: (0, 0)),
            pl.BlockSpec((1, H), lambda i: (0, 0)),
        ],
        out_specs=pl.BlockSpec((RB, H), lambda i: (i, 0)),
        out_shape=jax.ShapeDtypeStruct((N, H), jnp.float32),
    )(h, g.reshape(1, H), b.reshape(1, H))


def _xw_body(t_ref, W_ref, o_ref):
    o_ref[...] = jnp.dot(t_ref[...], W_ref[0],
                         preferred_element_type=jnp.float32)[None]


def _xw(t, W):
    return pl.pallas_call(
        _xw_body,
        grid=(NRB, R),
        in_specs=[
            pl.BlockSpec((RB, H), lambda i, r: (i, 0)),
            pl.BlockSpec((1, H, H), lambda i, r: (r, 0, 0)),
        ],
        out_specs=pl.BlockSpec((1, RB, H), lambda i, r: (r, i, 0)),
        out_shape=jax.ShapeDtypeStruct((R, N, H), jnp.float32),
    )(t, W)


def _combine_body(h_ref, agg_ref, t_ref, root_ref, bias_ref, o_ref):
    o_ref[...] = (h_ref[...] + agg_ref[...] + bias_ref[...]
                  + jnp.dot(t_ref[...], root_ref[...],
                            preferred_element_type=jnp.float32))


def _combine(h, agg, t, root, bias):
    return pl.pallas_call(
        _combine_body,
        grid=(NRB,),
        in_specs=[
            pl.BlockSpec((RB, H), lambda i: (i, 0)),
            pl.BlockSpec((RB, H), lambda i: (i, 0)),
            pl.BlockSpec((RB, H), lambda i: (i, 0)),
            pl.BlockSpec((H, H), lambda i: (0, 0)),
            pl.BlockSpec((1, H), lambda i: (0, 0)),
        ],
        out_specs=pl.BlockSpec((RB, H), lambda i: (i, 0)),
        out_shape=jax.ShapeDtypeStruct((N, H), jnp.float32),
    )(h, agg, t, root, bias.reshape(1, H))


def _post_body(h_ref, W_ref, b_ref, o_ref):
    o_ref[...] = jnp.dot(h_ref[...], W_ref[...],
                         preferred_element_type=jnp.float32) + b_ref[...]


def _post(h, W, b):
    return pl.pallas_call(
        _post_body,
        grid=(NRB,),
        in_specs=[
            pl.BlockSpec((RB, H), lambda i: (i, 0)),
            pl.BlockSpec((H, H), lambda i: (0, 0)),
            pl.BlockSpec((1, H), lambda i: (0, 0)),
        ],
        out_specs=pl.BlockSpec((RB, H), lambda i: (i, 0)),
        out_shape=jax.ShapeDtypeStruct((N, H), jnp.float32),
    )(h, W, b.reshape(1, H))


def _wid():
    return lax.axis_index("c") * 16 + lax.axis_index("s")


def _counts_body(dst_hbm, typ_hbm, out_hbm, dbuf, tbuf, hist):
    w = _wid()
    zero = jnp.zeros((16,), jnp.float32)
    lane = lax.iota(jnp.int32, 16)

    def zf(i, _):
        hist[pl.ds(i * 16, 16)] = zero
        return 0

    lax.fori_loop(0, NRPAD // 16, zf, 0)
    ept = E // NW
    pltpu.sync_copy(dst_hbm.at[pl.ds(w * ept, ept)], dbuf)
    pltpu.sync_copy(typ_hbm.at[pl.ds(w * ept, ept)], tbuf)

    def step(k, _):
        d16 = dbuf[pl.ds(k * 16, 16)]
        t16 = tbuf[pl.ds(k * 16, 16)]
        ridx = t16 * N + d16
        for b in range(16):
            idx = ridx[b]
            base = (idx >> 4) << 4
            onehot = jnp.where(lane == (idx & 15), 1.0, 0.0)
            plsc.addupdate(hist.at[pl.ds(base, 16)], onehot)
        return 0

    lax.fori_loop(0, ept // 16, step, 0)
    pltpu.sync_copy(hist, out_hbm.at[pl.ds(w * NRPAD, NRPAD)])


def _counts(dst, typ):
    f = pl.kernel(
        _counts_body,
        out_type=jax.ShapeDtypeStruct((NW * NRPAD,), jnp.float32),
        mesh=_SC_MESH,
        compiler_params=_SC_PARAMS,
        scratch_types=[
            pltpu.VMEM((E // NW,), jnp.int32),
            pltpu.VMEM((E // NW,), jnp.int32),
            pltpu.VMEM((NRPAD,), jnp.float32),
        ],
    )
    return f(dst, typ)


def _compact_body(src_hbm, dst_hbm, typ_hbm, cpack_hbm, nb_hbm,
                  sbufA, dbufA, tbufA, sbufB, dbufB, tbufB, cbuf, vbuf,
                  s1A, s2A, s3A, s1B, s2B, s3B):
    w = _wid()
    lane = lax.iota(jnp.int32, 16)
    lane16 = lane + 16
    dp16 = jnp.full((16,), DUMMYP, jnp.int32)
    for k in range(6):
        cbuf[pl.ds(k * 16, 16)] = dp16

    bufsA = (sbufA, dbufA, tbufA, s1A, s2A, s3A)
    bufsB = (sbufB, dbufB, tbufB, s1B, s2B, s3B)

    def issue(B, bufs):
        sbuf, dbuf, tbuf, s1, s2, s3 = bufs
        sl = pl.ds(B * SCAN, SCAN)
        pltpu.async_copy(src_hbm.at[sl], sbuf, s1)
        pltpu.async_copy(dst_hbm.at[sl], dbuf, s2)
        pltpu.async_copy(typ_hbm.at[sl], tbuf, s3)

    def consume(B, bufs, carry0):
        sbuf, dbuf, tbuf, s1, s2, s3 = bufs
        sl = pl.ds(B * SCAN, SCAN)
        pltpu.make_async_copy(src_hbm.at[sl], sbuf, s1).wait()
        pltpu.make_async_copy(dst_hbm.at[sl], dbuf, s2).wait()
        pltpu.make_async_copy(typ_hbm.at[sl], tbuf, s3).wait()

        def chunk(k, carry):
            cnt, blk = carry
            s16 = sbuf[pl.ds(k * 16, 16)]
            d16 = dbuf[pl.ds(k * 16, 16)]
            t16 = tbuf[pl.ds(k * 16, 16)]
            own = (d16 * 13108) >> 22
            m = own == w
            dloc = d16 - own * TS
            vp = jnp.where(m, s16 + (dloc << 14) + (t16 << 23), DUMMYP)
            key = jnp.where(m, lane, lane16)
            sp = plsc.sort_key_val(key, vp)[1]
            cbuf[pl.ds(cnt, 16)] = sp
            pc = plsc.all_reduce_population_count(m)
            cnt2 = cnt + pc[0]
            full = cnt2 >= 80

            @pl.when(full)
            def _():
                pltpu.sync_copy(
                    cbuf.at[pl.ds(0, 80)],
                    cpack_hbm.at[pl.ds(w * CAP + blk * 80, 80)])
                tv = cbuf[pl.ds(80, 16)]
                for k2 in range(6):
                    cbuf[pl.ds(k2 * 16, 16)] = dp16
                cbuf[pl.ds(0, 16)] = tv

            cnt3 = jnp.where(full, cnt2 - 80, cnt2)
            blk2 = jnp.where(full, blk + 1, blk)
            return (cnt3, blk2)

        return lax.fori_loop(0, SCAN // 16, chunk, carry0)

    nblocks = E // SCAN
    issue(0, bufsA)
    if nblocks > 1:
        issue(1, bufsB)
    carry = (jnp.int32(0), jnp.int32(0))
    for B in range(nblocks):
        bufs = bufsA if B % 2 == 0 else bufsB
        carry = consume(B, bufs, carry)
        if B + 2 < nblocks:
            issue(B + 2, bufs)
    cnt, blk = carry

    @pl.when(cnt > 0)
    def _():
        cbuf[pl.ds(cnt, 16)] = dp16
        pltpu.sync_copy(cbuf.at[pl.ds(0, 80)],
                        cpack_hbm.at[pl.ds(w * CAP + blk * 80, 80)])

    nb = blk + jnp.where(cnt > 0, 1, 0).astype(jnp.int32)
    vbuf[pl.ds(0, 16)] = jnp.where(lane == 0, nb, 0)
    pltpu.sync_copy(vbuf, nb_hbm.at[pl.ds(w * 16, 16)])


def _compact(src, dst, typ):
    f = pl.kernel(
        _compact_body,
        out_type=[
            jax.ShapeDtypeStruct((NW * CAP,), jnp.int32),
            jax.ShapeDtypeStruct((NW * 16,), jnp.int32),
        ],
        mesh=_SC_MESH,
        compiler_params=_SC_PARAMS,
        scratch_types=[
            pltpu.VMEM((SCAN,), jnp.int32),
            pltpu.VMEM((SCAN,), jnp.int32),
            pltpu.VMEM((SCAN,), jnp.int32),
            pltpu.VMEM((SCAN,), jnp.int32),
            pltpu.VMEM((SCAN,), jnp.int32),
            pltpu.VMEM((SCAN,), jnp.int32),
            pltpu.VMEM((96,), jnp.int32),
            pltpu.VMEM((16,), jnp.int32),
            pltpu.SemaphoreType.DMA,
            pltpu.SemaphoreType.DMA,
            pltpu.SemaphoreType.DMA,
            pltpu.SemaphoreType.DMA,
            pltpu.SemaphoreType.DMA,
            pltpu.SemaphoreType.DMA,
        ],
    )
    return f(src, dst, typ)


def _edge_body(cpack_hbm, nb_hbm, xw_hbm, inv_hbm, out_hbm,
               pbufA, ribufA, gbufA, ivalsA, rowsA,
               pbufB, ribufB, gbufB, ivalsB, rowsB,
               acc, nbb, semA, isemA, semB, isemB):
    w = _wid()
    zero = jnp.zeros((16,), jnp.float32)

    def zf(i, _):
        for q in range(H // 16):
            acc[i, pl.ds(q * 16, 16)] = zero
        return 0

    lax.fori_loop(0, TS + 1, zf, 0)
    pltpu.sync_copy(nb_hbm.at[pl.ds(w * 16, 16)], nbb)
    nb = nbb[pl.ds(0, 16)][0]

    bufsA = (pbufA, ribufA, gbufA, ivalsA, rowsA, semA, isemA)
    bufsB = (pbufB, ribufB, gbufB, ivalsB, rowsB, semB, isemB)

    def issue(b, bufs):
        pbuf, ribuf, gbuf, ivals, rows, sem, isem = bufs
        base = w * CAP + b * 80
        pltpu.sync_copy(cpack_hbm.at[pl.ds(base, 80)], pbuf)
        for a in range(5):
            sl = pl.ds(a * 16, 16)
            p = pbuf[sl]
            tN = (p >> 23) * N
            gbuf[sl] = tN + (p & 16383)
            ribuf[sl] = tN + ((p >> 14) & 511) + w * TS
        pltpu.async_copy(xw_hbm.at[gbuf], rows, sem)
        pltpu.async_copy(inv_hbm.at[ribuf], ivals, isem)

    def consume(bufs):
        pbuf, ribuf, gbuf, ivals, rows, sem, isem = bufs
        pltpu.make_async_copy(xw_hbm.at[gbuf], rows, sem).wait()
        pltpu.make_async_copy(inv_hbm.at[ribuf], ivals, isem).wait()

        def acc5(a, _):
            vp16 = pbuf[pl.ds(a * 16, 16)]
            dl16 = (vp16 >> 14) & 511
            iv16 = ivals[pl.ds(a * 16, 16)]
            for bb in range(16):
                j = dl16[bb]
                sp = jnp.full((16,), iv16[bb], jnp.float32)
                e = a * 16 + bb
                for q in range(H // 16):
                    sl = pl.ds(q * 16, 16)
                    plsc.addupdate(acc.at[j, sl], rows[e, sl] * sp)
            return 0

        lax.fori_loop(0, 5, acc5, 0)

    @pl.when(nb > 0)
    def _():
        issue(jnp.int32(0), bufsA)

    @pl.when(nb > 1)
    def _():
        issue(jnp.int32(1), bufsB)

    def batch(b, _):
        @pl.when((b & 1) == 0)
        def _():
            consume(bufsA)

            @pl.when(b + 2 < nb)
            def _():
                issue(b + 2, bufsA)

        @pl.when((b & 1) == 1)
        def _():
            consume(bufsB)

            @pl.when(b + 2 < nb)
            def _():
                issue(b + 2, bufsB)

        return 0

    lax.fori_loop(0, nb, batch, 0)

    @pl.when(w < NW - 1)
    def _():
        for q in range(4):
            pltpu.sync_copy(acc.at[pl.ds(q * 80, 80)],
                            out_hbm.at[pl.ds(w * TS + q * 80, 80)])

    @pl.when(w == NW - 1)
    def _():
        pltpu.sync_copy(acc.at[pl.ds(0, 80)],
                        out_hbm.at[pl.ds(w * TS, 80)])


def _edge(cpack, nbf, xwf, inv):
    f = pl.kernel(
        _edge_body,
        out_type=jax.ShapeDtypeStruct((N, H), jnp.float32),
        mesh=_SC_MESH,
        compiler_params=_SC_PARAMS,
        scratch_types=[
            pltpu.VMEM((80,), jnp.int32),
            pltpu.VMEM((80,), jnp.int32),
            pltpu.VMEM((80,), jnp.int32),
            pltpu.VMEM((80,), jnp.float32),
            pltpu.VMEM((80, H), jnp.float32),
            pltpu.VMEM((80,), jnp.int32),
            pltpu.VMEM((80,), jnp.int32),
            pltpu.VMEM((80,), jnp.int32),
            pltpu.VMEM((80,), jnp.float32),
            pltpu.VMEM((80, H), jnp.float32),
            pltpu.VMEM((TS + 1, H), jnp.float32),
            pltpu.VMEM((16,), jnp.int32),
            pltpu.SemaphoreType.DMA,
            pltpu.SemaphoreType.DMA,
            pltpu.SemaphoreType.DMA,
            pltpu.SemaphoreType.DMA,
        ],
    )
    return f(cpack, nbf, xwf, inv)


def kernel(x, bert_x, edge_index, edge_type, pre_W, pre_b, ie_W1, ie_b1,
           ie_g1, ie_bb1, ie_W2, ie_b2, ln_g0, ln_b0, basis0, comp0, root0,
           bias0, ln_g1, ln_b1, basis1, comp1, root1, bias1, post_W, post_b):
    src = edge_index[0]
    dst = edge_index[1]

    h = _fuse(x, bert_x, pre_W, pre_b, ie_W1, ie_b1, ie_g1, ie_bb1, ie_W2,
              ie_b2)
    cntp = _counts(dst, edge_type).reshape(NW, NRPAD)
    W0f, W1f, inv = _prep(comp0, basis0.reshape(NB, H * H), comp1,
                          basis1.reshape(NB, H * H), cntp)
    inv = inv.reshape(NRPAD)
    cpack, nbf = _compact(src, dst, edge_type)
    layers = [
        (ln_g0, ln_b0, W0f.reshape(R, H, H), root0, bias0),
        (ln_g1, ln_b1, W1f.reshape(R, H, H), root1, bias1),
    ]
    for (g, b, W, root, bias) in layers:
        t = _lnrelu(h, g, b)
        xwf = _xw(t, W).reshape(R * N, H)
        agg = _edge(cpack, nbf, xwf, inv)
        h = _combine(h, agg, t, root, bias)
    return _post(h, post_W, post_b)


# speedup vs baseline: 9.8320x; 1.0450x over previous
import jax
import jax.numpy as jnp
from jax import lax
from jax.experimental import pallas as pl
from jax.experimental.pallas import tpu as pltpu
from jax.experimental.pallas import tpu_sc as plsc

N = 10000
E = 320000
R = 8
NB = 4
D_IN = 128
H = 256
BERT = 768
IE = H + BERT

TS = 320
NW = 32
CAP = E + 80
NRPAD = 80128
DUMMYP = TS << 14
SCAN = 16000
RB = 400
NRB = N // RB

_SC_MESH = plsc.VectorSubcoreMesh(core_axis_name="c", subcore_axis_name="s")
_SC_PARAMS = pltpu.CompilerParams(needs_layout_passes=False)


def _fuse_body(x_ref, bert_ref, preW_ref, preb_ref, W1_ref, b1_ref, g1_ref,
               bb1_ref, W2_ref, b2_ref, out_ref):
    h = jnp.dot(x_ref[...], preW_ref[...], preferred_element_type=jnp.float32)
    h = jnp.maximum(h + preb_ref[...], 0.0)
    z = jnp.concatenate([h, bert_ref[...]], axis=1)
    u = jnp.dot(z, W1_ref[...], preferred_element_type=jnp.float32) + b1_ref[...]
    m = jnp.mean(u, axis=1, keepdims=True)
    v = jnp.mean((u - m) ** 2, axis=1, keepdims=True)
    u = (u - m) * lax.rsqrt(v + 1e-5) * g1_ref[...] + bb1_ref[...]
    u = jnp.maximum(u, 0.0)
    w = jnp.tanh(jnp.dot(u, W2_ref[...], preferred_element_type=jnp.float32)
                 + b2_ref[...])
    out_ref[...] = w[:, :H]


def _fuse(x, bert_x, preW, preb, W1, b1, g1, bb1, W2, b2):
    return pl.pallas_call(
        _fuse_body,
        grid=(NRB,),
        in_specs=[
            pl.BlockSpec((RB, D_IN), lambda i: (i, 0)),
            pl.BlockSpec((RB, BERT), lambda i: (i, 0)),
            pl.BlockSpec((D_IN, H), lambda i: (0, 0)),
            pl.BlockSpec((1, H), lambda i: (0, 0)),
            pl.BlockSpec((IE, IE), lambda i: (0, 0)),
            pl.BlockSpec((1, IE), lambda i: (0, 0)),
            pl.BlockSpec((1, IE), lambda i: (0, 0)),
            pl.BlockSpec((1, IE), lambda i: (0, 0)),
            pl.BlockSpec((IE, IE), lambda i: (0, 0)),
            pl.BlockSpec((1, IE), lambda i: (0, 0)),
        ],
        out_specs=pl.BlockSpec((RB, H), lambda i: (i, 0)),
        out_shape=jax.ShapeDtypeStruct((N, H), jnp.float32),
    )(x, bert_x, preW, preb.reshape(1, H), W1, b1.reshape(1, IE),
      g1.reshape(1, IE), bb1.reshape(1, IE), W2, b2.reshape(1, IE))


def _prep_body(c0_ref, b0_ref, c1_ref, b1_ref, cnt_ref, W0_ref, W1_ref,
               inv_ref):
    W0_ref[...] = jnp.dot(c0_ref[...], b0_ref[...],
                          preferred_element_type=jnp.float32)
    W1_ref[...] = jnp.dot(c1_ref[...], b1_ref[...],
                          preferred_element_type=jnp.float32)
    c = jnp.sum(cnt_ref[...], axis=0, keepdims=True)
    col = lax.broadcasted_iota(jnp.int32, (1, NRPAD), 1)
    inv_ref[...] = jnp.where(col < N * R, 1.0 / jnp.maximum(c, 1.0), 0.0)


def _prep(comp0, basis0f, comp1, basis1f, cnt):
    return pl.pallas_call(
        _prep_body,
        grid=(1,),
        in_specs=[
            pl.BlockSpec((R, NB), lambda i: (0, 0)),
            pl.BlockSpec((NB, H * H), lambda i: (0, 0)),
            pl.BlockSpec((R, NB), lambda i: (0, 0)),
            pl.BlockSpec((NB, H * H), lambda i: (0, 0)),
            pl.BlockSpec((NW, NRPAD), lambda i: (0, 0)),
        ],
        out_specs=[
            pl.BlockSpec((R, H * H), lambda i: (0, 0)),
            pl.BlockSpec((R, H * H), lambda i: (0, 0)),
            pl.BlockSpec((1, NRPAD), lambda i: (0, 0)),
        ],
        out_shape=[
            jax.ShapeDtypeStruct((R, H * H), jnp.float32),
            jax.ShapeDtypeStruct((R, H * H), jnp.float32),
            jax.ShapeDtypeStruct((1, NRPAD), jnp.float32),
        ],
    )(comp0, basis0f, comp1, basis1f, cnt)


def _lnrelu_body(h_ref, g_ref, b_ref, o_ref):
    h = h_ref[...]
    m = jnp.mean(h, axis=1, keepdims=True)
    v = jnp.mean((h - m) ** 2, axis=1, keepdims=True)
    o_ref[...] = jnp.maximum((h - m) * lax.rsqrt(v + 1e-5) * g_ref[...]
                             + b_ref[...], 0.0)


def _lnrelu(h, g, b):
    return pl.pallas_call(
        _lnrelu_body,
        grid=(NRB,),
        in_specs=[
            pl.BlockSpec((RB, H), lambda i: (i, 0)),
            pl.BlockSpec((1, H), lambda i: (0, 0)),
            pl.BlockSpec((1, H), lambda i: (0, 0)),
        ],
        out_specs=pl.BlockSpec((RB, H), lambda i: (i, 0)),
        out_shape=jax.ShapeDtypeStruct((N, H), jnp.float32),
    )(h, g.reshape(1, H), b.reshape(1, H))


def _xw_body(t_ref, W_ref, o_ref):
    o_ref[...] = jnp.dot(t_ref[...], W_ref[0],
                         preferred_element_type=jnp.float32)[None]


def _xw(t, W):
    return pl.pallas_call(
        _xw_body,
        grid=(NRB, R),
        in_specs=[
            pl.BlockSpec((RB, H), lambda i, r: (i, 0)),
            pl.BlockSpec((1, H, H), lambda i, r: (r, 0, 0)),
        ],
        out_specs=pl.BlockSpec((1, RB, H), lambda i, r: (r, i, 0)),
        out_shape=jax.ShapeDtypeStruct((R, N, H), jnp.float32),
    )(t, W)


def _combine_body(h_ref, agg_ref, t_ref, root_ref, bias_ref, o_ref):
    o_ref[...] = (h_ref[...] + agg_ref[...] + bias_ref[...]
                  + jnp.dot(t_ref[...], root_ref[...],
                            preferred_element_type=jnp.float32))


def _combine(h, agg, t, root, bias):
    return pl.pallas_call(
        _combine_body,
        grid=(NRB,),
        in_specs=[
            pl.BlockSpec((RB, H), lambda i: (i, 0)),
            pl.BlockSpec((RB, H), lambda i: (i, 0)),
            pl.BlockSpec((RB, H), lambda i: (i, 0)),
            pl.BlockSpec((H, H), lambda i: (0, 0)),
            pl.BlockSpec((1, H), lambda i: (0, 0)),
        ],
        out_specs=pl.BlockSpec((RB, H), lambda i: (i, 0)),
        out_shape=jax.ShapeDtypeStruct((N, H), jnp.float32),
    )(h, agg, t, root, bias.reshape(1, H))


def _post_body(h_ref, W_ref, b_ref, o_ref):
    o_ref[...] = jnp.dot(h_ref[...], W_ref[...],
                         preferred_element_type=jnp.float32) + b_ref[...]


def _post(h, W, b):
    return pl.pallas_call(
        _post_body,
        grid=(NRB,),
        in_specs=[
            pl.BlockSpec((RB, H), lambda i: (i, 0)),
            pl.BlockSpec((H, H), lambda i: (0, 0)),
            pl.BlockSpec((1, H), lambda i: (0, 0)),
        ],
        out_specs=pl.BlockSpec((RB, H), lambda i: (i, 0)),
        out_shape=jax.ShapeDtypeStruct((N, H), jnp.float32),
    )(h, W, b.reshape(1, H))


def _wid():
    return lax.axis_index("c") * 16 + lax.axis_index("s")


def _counts_body(dst_hbm, typ_hbm, out_hbm, dbuf, tbuf, hist):
    w = _wid()
    zero = jnp.zeros((16,), jnp.float32)
    lane = lax.iota(jnp.int32, 16)

    def zf(i, _):
        hist[pl.ds(i * 16, 16)] = zero
        return 0

    lax.fori_loop(0, NRPAD // 16, zf, 0)
    ept = E // NW
    pltpu.sync_copy(dst_hbm.at[pl.ds(w * ept, ept)], dbuf)
    pltpu.sync_copy(typ_hbm.at[pl.ds(w * ept, ept)], tbuf)

    def step(k, _):
        d16 = dbuf[pl.ds(k * 16, 16)]
        t16 = tbuf[pl.ds(k * 16, 16)]
        ridx = t16 * N + d16
        for b in range(16):
            idx = ridx[b]
            base = (idx >> 4) << 4
            onehot = jnp.where(lane == (idx & 15), 1.0, 0.0)
            plsc.addupdate(hist.at[pl.ds(base, 16)], onehot)
        return 0

    lax.fori_loop(0, ept // 16, step, 0)
    pltpu.sync_copy(hist, out_hbm.at[pl.ds(w * NRPAD, NRPAD)])


def _counts(dst, typ):
    f = pl.kernel(
        _counts_body,
        out_type=jax.ShapeDtypeStruct((NW * NRPAD,), jnp.float32),
        mesh=_SC_MESH,
        compiler_params=_SC_PARAMS,
        scratch_types=[
            pltpu.VMEM((E // NW,), jnp.int32),
            pltpu.VMEM((E // NW,), jnp.int32),
            pltpu.VMEM((NRPAD,), jnp.float32),
        ],
    )
    return f(dst, typ)


def _compact_body(src_hbm, dst_hbm, typ_hbm, cpack_hbm, nb_hbm,
                  sbufA, dbufA, tbufA, sbufB, dbufB, tbufB, cbuf, vbuf,
                  s1A, s2A, s3A, s1B, s2B, s3B):
    w = _wid()
    lane = lax.iota(jnp.int32, 16)
    lane16 = lane + 16
    dp16 = jnp.full((16,), DUMMYP, jnp.int32)
    for k in range(6):
        cbuf[pl.ds(k * 16, 16)] = dp16

    bufsA = (sbufA, dbufA, tbufA, s1A, s2A, s3A)
    bufsB = (sbufB, dbufB, tbufB, s1B, s2B, s3B)

    def issue(B, bufs):
        sbuf, dbuf, tbuf, s1, s2, s3 = bufs
        sl = pl.ds(B * SCAN, SCAN)
        pltpu.async_copy(src_hbm.at[sl], sbuf, s1)
        pltpu.async_copy(dst_hbm.at[sl], dbuf, s2)
        pltpu.async_copy(typ_hbm.at[sl], tbuf, s3)

    def consume(B, bufs, carry0):
        sbuf, dbuf, tbuf, s1, s2, s3 = bufs
        sl = pl.ds(B * SCAN, SCAN)
        pltpu.make_async_copy(src_hbm.at[sl], sbuf, s1).wait()
        pltpu.make_async_copy(dst_hbm.at[sl], dbuf, s2).wait()
        pltpu.make_async_copy(typ_hbm.at[sl], tbuf, s3).wait()

        def chunk(k, carry):
            cnt, blk = carry
            s16 = sbuf[pl.ds(k * 16, 16)]
            d16 = dbuf[pl.ds(k * 16, 16)]
            t16 = tbuf[pl.ds(k * 16, 16)]
            own = (d16 * 13108) >> 22
            m = own == w
            dloc = d16 - own * TS
            vp = jnp.where(m, s16 + (dloc << 14) + (t16 << 23), DUMMYP)
            key = jnp.where(m, lane, lane16)
            sp = plsc.sort_key_val(key, vp)[1]
            cbuf[pl.ds(cnt, 16)] = sp
            pc = plsc.all_reduce_population_count(m)
            cnt2 = cnt + pc[0]
            full = cnt2 >= 80

            @pl.when(full)
            def _():
                pltpu.sync_copy(
                    cbuf.at[pl.ds(0, 80)],
                    cpack_hbm.at[pl.ds(w * CAP + blk * 80, 80)])
                tv = cbuf[pl.ds(80, 16)]
                for k2 in range(6):
                    cbuf[pl.ds(k2 * 16, 16)] = dp16
                cbuf[pl.ds(0, 16)] = tv

            cnt3 = jnp.where(full, cnt2 - 80, cnt2)
            blk2 = jnp.where(full, blk + 1, blk)
            return (cnt3, blk2)

        return lax.fori_loop(0, SCAN // 16, chunk, carry0)

    nblocks = E // SCAN
    issue(0, bufsA)
    if nblocks > 1:
        issue(1, bufsB)
    carry = (jnp.int32(0), jnp.int32(0))
    for B in range(nblocks):
        bufs = bufsA if B % 2 == 0 else bufsB
        carry = consume(B, bufs, carry)
        if B + 2 < nblocks:
            issue(B + 2, bufs)
    cnt, blk = carry

    @pl.when(cnt > 0)
    def _():
        cbuf[pl.ds(cnt, 16)] = dp16
        pltpu.sync_copy(cbuf.at[pl.ds(0, 80)],
                        cpack_hbm.at[pl.ds(w * CAP + blk * 80, 80)])

    nb = blk + jnp.where(cnt > 0, 1, 0).astype(jnp.int32)
    vbuf[pl.ds(0, 16)] = jnp.where(lane == 0, nb, 0)
    pltpu.sync_copy(vbuf, nb_hbm.at[pl.ds(w * 16, 16)])


def _compact(src, dst, typ):
    f = pl.kernel(
        _compact_body,
        out_type=[
            jax.ShapeDtypeStruct((NW * CAP,), jnp.int32),
            jax.ShapeDtypeStruct((NW * 16,), jnp.int32),
        ],
        mesh=_SC_MESH,
        compiler_params=_SC_PARAMS,
        scratch_types=[
            pltpu.VMEM((SCAN,), jnp.int32),
            pltpu.VMEM((SCAN,), jnp.int32),
            pltpu.VMEM((SCAN,), jnp.int32),
            pltpu.VMEM((SCAN,), jnp.int32),
            pltpu.VMEM((SCAN,), jnp.int32),
            pltpu.VMEM((SCAN,), jnp.int32),
            pltpu.VMEM((96,), jnp.int32),
            pltpu.VMEM((16,), jnp.int32),
            pltpu.SemaphoreType.DMA,
            pltpu.SemaphoreType.DMA,
            pltpu.SemaphoreType.DMA,
            pltpu.SemaphoreType.DMA,
            pltpu.SemaphoreType.DMA,
            pltpu.SemaphoreType.DMA,
        ],
    )
    return f(src, dst, typ)


def _edge_body(cpack_hbm, nb_hbm, xw_hbm, inv_hbm, out_hbm,
               pbufA, ribufA, gbufA, dlbufA, ivalsA, rowsA,
               pbufB, ribufB, gbufB, dlbufB, ivalsB, rowsB,
               acc, nbb, semA, isemA, psemA, semB, isemB, psemB):
    w = _wid()
    zero = jnp.zeros((16,), jnp.float32)

    def zf(i, _):
        for q in range(H // 16):
            acc[i, pl.ds(q * 16, 16)] = zero
        return 0

    lax.fori_loop(0, TS + 1, zf, 0)
    pltpu.sync_copy(nb_hbm.at[pl.ds(w * 16, 16)], nbb)
    nb = nbb[pl.ds(0, 16)][0]

    bufsA = (pbufA, ribufA, gbufA, dlbufA, ivalsA, rowsA, semA, isemA, psemA)
    bufsB = (pbufB, ribufB, gbufB, dlbufB, ivalsB, rowsB, semB, isemB, psemB)

    def pload(b, bufs):
        pbuf, psem = bufs[0], bufs[8]
        base = w * CAP + b * 80
        pltpu.async_copy(cpack_hbm.at[pl.ds(base, 80)], pbuf, psem)

    def issue(b, bufs):
        pbuf, ribuf, gbuf, dlbuf, ivals, rows, sem, isem, psem = bufs
        base = w * CAP + b * 80
        pltpu.make_async_copy(cpack_hbm.at[pl.ds(base, 80)], pbuf,
                              psem).wait()
        for a in range(5):
            sl = pl.ds(a * 16, 16)
            p = pbuf[sl]
            tN = (p >> 23) * N
            gbuf[sl] = tN + (p & 16383)
            ribuf[sl] = tN + ((p >> 14) & 511) + w * TS
            dlbuf[sl] = (p >> 14) & 511
        pltpu.async_copy(xw_hbm.at[gbuf], rows, sem)
        pltpu.async_copy(inv_hbm.at[ribuf], ivals, isem)

    def consume(bufs):
        pbuf, ribuf, gbuf, dlbuf, ivals, rows, sem, isem, psem = bufs
        pltpu.make_async_copy(xw_hbm.at[gbuf], rows, sem).wait()
        pltpu.make_async_copy(inv_hbm.at[ribuf], ivals, isem).wait()

        def acc5(a, _):
            dl16 = dlbuf[pl.ds(a * 16, 16)]
            iv16 = ivals[pl.ds(a * 16, 16)]
            for bb in range(16):
                j = dl16[bb]
                sp = jnp.full((16,), iv16[bb], jnp.float32)
                e = a * 16 + bb
                for q in range(H // 16):
                    sl = pl.ds(q * 16, 16)
                    plsc.addupdate(acc.at[j, sl], rows[e, sl] * sp)
            return 0

        lax.fori_loop(0, 5, acc5, 0)

    @pl.when(nb > 0)
    def _():
        pload(jnp.int32(0), bufsA)
        issue(jnp.int32(0), bufsA)

    @pl.when(nb > 1)
    def _():
        pload(jnp.int32(1), bufsB)
        issue(jnp.int32(1), bufsB)

    def batch(b, _):
        @pl.when((b & 1) == 0)
        def _():
            @pl.when(b + 2 < nb)
            def _():
                pload(b + 2, bufsA)

            consume(bufsA)

            @pl.when(b + 2 < nb)
            def _():
                issue(b + 2, bufsA)

        @pl.when((b & 1) == 1)
        def _():
            @pl.when(b + 2 < nb)
            def _():
                pload(b + 2, bufsB)

            consume(bufsB)

            @pl.when(b + 2 < nb)
            def _():
                issue(b + 2, bufsB)

        return 0

    lax.fori_loop(0, nb, batch, 0)

    @pl.when(w < NW - 1)
    def _():
        for q in range(4):
            pltpu.sync_copy(acc.at[pl.ds(q * 80, 80)],
                            out_hbm.at[pl.ds(w * TS + q * 80, 80)])

    @pl.when(w == NW - 1)
    def _():
        pltpu.sync_copy(acc.at[pl.ds(0, 80)],
                        out_hbm.at[pl.ds(w * TS, 80)])


def _edge(cpack, nbf, xwf, inv):
    f = pl.kernel(
        _edge_body,
        out_type=jax.ShapeDtypeStruct((N, H), jnp.float32),
        mesh=_SC_MESH,
        compiler_params=_SC_PARAMS,
        scratch_types=[
            pltpu.VMEM((80,), jnp.int32),
            pltpu.VMEM((80,), jnp.int32),
            pltpu.VMEM((80,), jnp.int32),
            pltpu.VMEM((80,), jnp.int32),
            pltpu.VMEM((80,), jnp.float32),
            pltpu.VMEM((80, H), jnp.float32),
            pltpu.VMEM((80,), jnp.int32),
            pltpu.VMEM((80,), jnp.int32),
            pltpu.VMEM((80,), jnp.int32),
            pltpu.VMEM((80,), jnp.int32),
            pltpu.VMEM((80,), jnp.float32),
            pltpu.VMEM((80, H), jnp.float32),
            pltpu.VMEM((TS + 1, H), jnp.float32),
            pltpu.VMEM((16,), jnp.int32),
            pltpu.SemaphoreType.DMA,
            pltpu.SemaphoreType.DMA,
            pltpu.SemaphoreType.DMA,
            pltpu.SemaphoreType.DMA,
            pltpu.SemaphoreType.DMA,
            pltpu.SemaphoreType.DMA,
        ],
    )
    return f(cpack, nbf, xwf, inv)


def kernel(x, bert_x, edge_index, edge_type, pre_W, pre_b, ie_W1, ie_b1,
           ie_g1, ie_bb1, ie_W2, ie_b2, ln_g0, ln_b0, basis0, comp0, root0,
           bias0, ln_g1, ln_b1, basis1, comp1, root1, bias1, post_W, post_b):
    src = edge_index[0]
    dst = edge_index[1]

    h = _fuse(x, bert_x, pre_W, pre_b, ie_W1, ie_b1, ie_g1, ie_bb1, ie_W2,
              ie_b2)
    cntp = _counts(dst, edge_type).reshape(NW, NRPAD)
    W0f, W1f, inv = _prep(comp0, basis0.reshape(NB, H * H), comp1,
                          basis1.reshape(NB, H * H), cntp)
    inv = inv.reshape(NRPAD)
    cpack, nbf = _compact(src, dst, edge_type)
    layers = [
        (ln_g0, ln_b0, W0f.reshape(R, H, H), root0, bias0),
        (ln_g1, ln_b1, W1f.reshape(R, H, H), root1, bias1),
    ]
    for (g, b, W, root, bias) in layers:
        t = _lnrelu(h, g, b)
        xwf = _xw(t, W).reshape(R * N, H)
        agg = _edge(cpack, nbf, xwf, inv)
        h = _combine(h, agg, t, root, bias)
    return _post(h, post_W, post_b)
